# baseline (speedup 1.0000x reference)
import jax
import jax.numpy as jnp
from jax.experimental import pallas as pl
from jax.experimental.pallas import tpu as pltpu

_H = 32
_W = 32
_R = 1


_BB = 2


def _buddy_kernel(cue_ref, patches_ref, out_ref):
    for i in range(_BB):
        patches = patches_ref[i]
        cue = cue_ref[i]
        sim = jax.lax.dot_general(
            cue, patches, (((1,), (1,)), ((), ())),
            preferred_element_type=jnp.float32)
        idx = jnp.argmax(sim, axis=1)
        K = cue.shape[0]
        for k in range(K):
            h = idx[k] // _W
            w = idx[k] % _W
            acc = jnp.zeros((1, patches.shape[1]), jnp.float32)
            cnt = 0.0
            for dh in (-1, 0, 1):
                for dw in (-1, 0, 1):
                    hh = h + dh
                    ww = w + dw
                    valid = ((hh >= 0) & (hh < _H) & (ww >= 0) & (ww < _W))
                    pos = (jnp.clip(hh, 0, _H - 1) * _W
                           + jnp.clip(ww, 0, _W - 1))
                    row = patches_ref[i, pl.ds(pos, 1), :]
                    vf = valid.astype(jnp.float32)
                    acc = acc + row * vf
                    cnt = cnt + vf
            out_ref[i, pl.ds(k, 1), :] = acc / cnt


def kernel(cue, patches):
    B, K, D = cue.shape
    _, H, W, _ = patches.shape
    patches_flat = patches.reshape(B, H * W, D)
    return pl.pallas_call(
        _buddy_kernel,
        grid=(B // _BB,),
        in_specs=[
            pl.BlockSpec((_BB, K, D), lambda b: (b, 0, 0)),
            pl.BlockSpec((_BB, H * W, D), lambda b: (b, 0, 0)),
        ],
        out_specs=pl.BlockSpec((_BB, K, D), lambda b: (b, 0, 0)),
        out_shape=jax.ShapeDtypeStruct((B, K, D), jnp.float32),
        compiler_params=pltpu.CompilerParams(
            dimension_semantics=("parallel",)),
    )(cue, patches_flat)


# speedup vs baseline: 1.1738x; 1.1738x over previous
import jax
import jax.numpy as jnp
from jax.experimental import pallas as pl
from jax.experimental.pallas import tpu as pltpu

_H = 32
_W = 32
_R = 1


_BB = 4


def _buddy_kernel(cue_ref, patches_ref, out_ref):
    for i in range(_BB):
        patches = patches_ref[i]
        cue = cue_ref[i]
        sim = jax.lax.dot_general(
            cue, patches, (((1,), (1,)), ((), ())),
            preferred_element_type=jnp.float32)
        idx = jnp.argmax(sim, axis=1)
        K = cue.shape[0]
        for k in range(K):
            h = idx[k] // _W
            w = idx[k] % _W
            acc = jnp.zeros((1, patches.shape[1]), jnp.float32)
            cnt = 0.0
            for dh in (-1, 0, 1):
                for dw in (-1, 0, 1):
                    hh = h + dh
                    ww = w + dw
                    valid = ((hh >= 0) & (hh < _H) & (ww >= 0) & (ww < _W))
                    pos = (jnp.clip(hh, 0, _H - 1) * _W
                           + jnp.clip(ww, 0, _W - 1))
                    row = patches_ref[i, pl.ds(pos, 1), :]
                    vf = valid.astype(jnp.float32)
                    acc = acc + row * vf
                    cnt = cnt + vf
            out_ref[i, pl.ds(k, 1), :] = acc * (1.0 / cnt)


def kernel(cue, patches):
    B, K, D = cue.shape
    _, H, W, _ = patches.shape
    patches_flat = patches.reshape(B, H * W, D)
    return pl.pallas_call(
        _buddy_kernel,
        grid=(B // _BB,),
        in_specs=[
            pl.BlockSpec((_BB, K, D), lambda b: (b, 0, 0)),
            pl.BlockSpec((_BB, H * W, D), lambda b: (b, 0, 0)),
        ],
        out_specs=pl.BlockSpec((_BB, K, D), lambda b: (b, 0, 0)),
        out_shape=jax.ShapeDtypeStruct((B, K, D), jnp.float32),
        compiler_params=pltpu.CompilerParams(
            dimension_semantics=("parallel",)),
    )(cue, patches_flat)


# speedup vs baseline: 1.1740x; 1.0001x over previous
import jax
import jax.numpy as jnp
from jax.experimental import pallas as pl
from jax.experimental.pallas import tpu as pltpu

_H = 32
_W = 32
_R = 1


_BB = 4


def _buddy_kernel(cue_ref, patches_ref, out_ref):
    for i in range(_BB):
        patches = patches_ref[i]
        cue = cue_ref[i]
        sim = jax.lax.dot_general(
            cue, patches, (((1,), (1,)), ((), ())),
            preferred_element_type=jnp.float32)
        idx = jnp.argmax(sim, axis=1)
        K = cue.shape[0]
        for k in range(K):
            h = idx[k] // _W
            w = idx[k] % _W
            hm = jnp.maximum(h - 1, 0)
            hp = jnp.minimum(h + 1, _H - 1)
            wm = jnp.maximum(w - 1, 0)
            wp = jnp.minimum(w + 1, _W - 1)
            vt = (h > 0).astype(jnp.float32)
            vb = (h < _H - 1).astype(jnp.float32)
            vl = (w > 0).astype(jnp.float32)
            vr = (w < _W - 1).astype(jnp.float32)
            cnt = (1.0 + vt + vb) * (1.0 + vl + vr)
            acc = patches_ref[i, pl.ds(h * _W + w, 1), :]
            for hr, vh in ((hm, vt), (h, None), (hp, vb)):
                base = hr * _W
                for wc, vw in ((wm, vl), (w, None), (wp, vr)):
                    if vh is None and vw is None:
                        continue
                    wt = (vh if vw is None else
                          vw if vh is None else vh * vw)
                    row = patches_ref[i, pl.ds(base + wc, 1), :]
                    acc = acc + row * wt
            out_ref[i, pl.ds(k, 1), :] = acc * (1.0 / cnt)


def kernel(cue, patches):
    B, K, D = cue.shape
    _, H, W, _ = patches.shape
    patches_flat = patches.reshape(B, H * W, D)
    return pl.pallas_call(
        _buddy_kernel,
        grid=(B // _BB,),
        in_specs=[
            pl.BlockSpec((_BB, K, D), lambda b: (b, 0, 0)),
            pl.BlockSpec((_BB, H * W, D), lambda b: (b, 0, 0)),
        ],
        out_specs=pl.BlockSpec((_BB, K, D), lambda b: (b, 0, 0)),
        out_shape=jax.ShapeDtypeStruct((B, K, D), jnp.float32),
        compiler_params=pltpu.CompilerParams(
            dimension_semantics=("parallel",)),
    )(cue, patches_flat)
